# Initial kernel scaffold; baseline (speedup 1.0000x reference)
import jax
import jax.numpy as jnp
from jax.experimental import pallas as pl


def kernel(x_nchw, conv_weight_oihw, conv_bias):
    raise NotImplementedError("write your pallas kernel here")


# speedup vs baseline: 1.0192x; 1.0192x over previous
import functools

import jax
import jax.numpy as jnp
import numpy as np
from jax.experimental import pallas as pl
from jax.experimental.pallas import tpu as pltpu

_FOLD = np.array([[[1, 0, 0], [0, 1, 1]],
                  [[1, 1, 0], [0, 0, 1]]], dtype=np.float32)


def _fold_weights(w_oihw):
    w = jnp.transpose(w_oihw, (2, 3, 1, 0))
    fold = jnp.asarray(_FOLD)
    return jnp.einsum("apk,bql,klio->abpqio", fold, fold, w)


def _conv_body(xp_ref, w_ref, b_ref, o_ref, *, TH, W, Cin, Cout):
    M = TH * W
    bias_v = b_ref[...]

    win = {}
    for p in range(3):
        for q in range(3):
            win[(p, q)] = xp_ref[0, pl.ds(p, TH), pl.ds(q, W), :].reshape(M, Cin)

    for a in range(2):
        for b in range(2):
            acc = None
            for dy in range(2):
                for dx in range(2):
                    d = jnp.dot(win[(a + dy, b + dx)], w_ref[a, b, dy, dx],
                                preferred_element_type=jnp.float32)
                    acc = d if acc is None else acc + d
            acc = acc + bias_v
            o_ref[0, a, b] = acc.reshape(TH, W, Cout).astype(o_ref.dtype)


def kernel(x_nchw, conv_weight_oihw, conv_bias):
    N, C, H, W = x_nchw.shape
    Cout = conv_weight_oihw.shape[0]
    TH = H

    x = jnp.transpose(x_nchw, (0, 2, 3, 1))
    x_pad = jnp.pad(x, ((0, 0), (1, 1), (1, 1), (0, 0))).astype(jnp.bfloat16)
    w_eff = _fold_weights(conv_weight_oihw).astype(jnp.bfloat16)
    bias2 = conv_bias.reshape(1, Cout).astype(jnp.float32)

    body = functools.partial(_conv_body, TH=TH, W=W, Cin=C, Cout=Cout)
    y_sub = pl.pallas_call(
        body,
        out_shape=jax.ShapeDtypeStruct((N, 2, 2, H, W, Cout), jnp.float32),
        grid=(N,),
        in_specs=[
            pl.BlockSpec((1, H + 2, W + 2, C), lambda n: (n, 0, 0, 0)),
            pl.BlockSpec((2, 2, 2, 2, C, Cout), lambda n: (0, 0, 0, 0, 0, 0)),
            pl.BlockSpec((1, Cout), lambda n: (0, 0)),
        ],
        out_specs=pl.BlockSpec((1, 2, 2, TH, W, Cout), lambda n: (n, 0, 0, 0, 0, 0)),
        compiler_params=pltpu.CompilerParams(
            dimension_semantics=("parallel",)),
        cost_estimate=pl.CostEstimate(
            flops=int(2 * 16 * N * H * W * C * Cout),
            transcendentals=0,
            bytes_accessed=int(N * (H + 2) * (W + 2) * C * 2
                               + N * 4 * H * W * Cout * 4),
        ),
    )(x_pad, w_eff, bias2)

    y = jnp.transpose(y_sub, (0, 5, 3, 1, 4, 2))
    return y.reshape(N, Cout, 2 * H, 2 * W)
